# baseline (speedup 1.0000x reference)
import functools

import jax
import jax.numpy as jnp
from jax import lax
from jax.experimental import pallas as pl
from jax.experimental.pallas import tpu as pltpu
from jax.experimental.pallas import tpu_sc as plsc

K = 16
NVAL = 8
CH = 16
LW = 128
L = 16
NTILES = 32
RB = 64


def _tc_body(block_q, nch, d_ref, st_ref, dist_ref, minis_ref):
    d = d_ref[...]
    st = st_ref[...]
    d_sq = jnp.sum(d * d, axis=1, keepdims=True)
    s_sq = jnp.sum(st * st, axis=0, keepdims=True)
    dot = lax.dot_general(d, st, (((1,), (0,)), ((), ())),
                          preferred_element_type=jnp.float32)
    dist = d_sq + s_sq - 2.0 * dot
    dist_ref[...] = dist
    minis_ref[...] = jnp.min(dist.reshape(block_q, nch, CH), axis=2)


def _merge(a, b):
    ka, va = a
    kb, vb = b
    rkb = lax.rev(kb, (0,))
    rvb = lax.rev(vb, (0,))
    take = ka <= rkb
    mk = jnp.minimum(ka, rkb)
    mv = jnp.where(take, va, rvb)
    return plsc.sort_key_val(mk, mv)


def _tree(pairs):
    while len(pairs) > 1:
        pairs = [_merge(pairs[2 * i], pairs[2 * i + 1])
                 for i in range(len(pairs) // 2)]
    return pairs[0]


def _sc_body(rows, nch, d_mask, mkeys,
             dist_ref, minis_ref, smt_ref, out_ref,
             smt_v, minis_v, out_v, hc_v, idx_v, cand_v, hist_v, sem, osem):
    nvpr = nch // L
    rpg = mkeys // LW
    wid = lax.axis_index("s") * 2 + lax.axis_index("c")
    base = wid * rows

    pltpu.sync_copy(smt_ref, smt_v)

    iota = lax.iota(jnp.int32, L)
    lane_lt8 = iota < NVAL

    def blk_body(b, carry0):
        pltpu.sync_copy(
            minis_ref.at[pl.ds((base + b * RB) * nch, RB * nch)], minis_v)

        def row_body(rl, carry):
            r = b * RB + rl
            pairs = [plsc.sort_key_val(minis_v[pl.ds(rl * nch + j * L, L)],
                                       iota + j * L)
                     for j in range(nvpr)]
            _mk, hc = _tree(pairs)
            hc_v[...] = hc

            idx_v[...] = (base + r) * rpg + (hc >> 3)
            pltpu.async_copy(dist_ref.at[idx_v], cand_v, sem).wait()

            hco = (hc & 7) * CH
            epairs = []
            for s in range(K):
                keys = cand_v[s, pl.ds(hco[s], CH)]
                epairs.append(plsc.sort_key_val(keys, iota + s * L))
            _ek, p = _tree(epairs)

            chunk = plsc.load_gather(hc_v, [p >> 4])
            orig = chunk * CH + (p & (CH - 1))

            mode_vec = jnp.zeros((L,), jnp.float32)
            for d in range(d_mask):
                vals = plsc.load_gather(
                    smt_v, [jnp.full((L,), d, jnp.int32), orig])
                vi = vals.astype(jnp.int32)
                hist_v[...] = jnp.zeros((L,), jnp.float32)
                plsc.addupdate_scatter(hist_v, [vi],
                                       jnp.ones((L,), jnp.float32))
                cnt = hist_v[...].astype(jnp.int32)
                score = cnt * NVAL + (NVAL - 1 - iota)
                best = jnp.max(score)
                mode = (NVAL - 1 - best % NVAL).astype(jnp.float32)
                mode_vec = jnp.where(iota == d, mode, mode_vec)

            plsc.store_scatter(out_v, [r * NVAL + iota], mode_vec,
                               mask=lane_lt8)
            return carry

        lax.fori_loop(0, RB, row_body, None)
        return carry0

    lax.fori_loop(0, rows // RB, blk_body, None)
    pltpu.async_copy(out_v, out_ref.at[pl.ds(base * NVAL, rows * NVAL)],
                     osem).wait()


def kernel(s_coor, s_mask, d_coor):
    mkeys, dim = s_coor.shape
    n = d_coor.shape[0]
    d_mask = s_mask.shape[1]
    nch = mkeys // CH
    block_q = 128 if n % 128 == 0 else n
    grid = n // block_q
    rows = n // NTILES

    s_t = s_coor.T
    smt = s_mask.T

    dist, minis = pl.pallas_call(
        functools.partial(_tc_body, block_q, nch),
        grid=(grid,),
        in_specs=[
            pl.BlockSpec((block_q, dim), lambda i: (i, 0)),
            pl.BlockSpec((dim, mkeys), lambda i: (0, 0)),
        ],
        out_specs=[
            pl.BlockSpec((block_q, mkeys), lambda i: (i, 0)),
            pl.BlockSpec((block_q, nch), lambda i: (i, 0)),
        ],
        out_shape=[
            jax.ShapeDtypeStruct((n, mkeys), jnp.float32),
            jax.ShapeDtypeStruct((n, nch), jnp.float32),
        ],
    )(d_coor, s_t)

    dist_rows = dist.reshape(n * mkeys // LW, LW)
    minis_flat = minis.reshape(n * nch)

    mesh = plsc.VectorSubcoreMesh(core_axis_name="c", subcore_axis_name="s",
                                  num_cores=2, num_subcores=16)
    sc = pl.kernel(
        functools.partial(_sc_body, rows, nch, d_mask, mkeys),
        out_type=jax.ShapeDtypeStruct((n * NVAL,), jnp.float32),
        mesh=mesh,
        compiler_params=pltpu.CompilerParams(needs_layout_passes=False),
        scratch_types=[
            pltpu.VMEM((d_mask, mkeys), jnp.float32),
            pltpu.VMEM((RB * nch,), jnp.float32),
            pltpu.VMEM((rows * NVAL,), jnp.float32),
            pltpu.VMEM((K,), jnp.int32),
            pltpu.VMEM((K,), jnp.int32),
            pltpu.VMEM((K, LW), jnp.float32),
            pltpu.VMEM((L,), jnp.float32),
            pltpu.SemaphoreType.DMA,
            pltpu.SemaphoreType.DMA,
        ],
    )
    out = sc(dist_rows, minis_flat, smt)
    return out.reshape(n, NVAL)


# speedup vs baseline: 1.1638x; 1.1638x over previous
import functools

import jax
import jax.numpy as jnp
from jax import lax
from jax.experimental import pallas as pl
from jax.experimental.pallas import tpu as pltpu
from jax.experimental.pallas import tpu_sc as plsc

K = 16
NVAL = 8
CH = 16
LW = 128
L = 16
NTILES = 32
RB = 64


def _tc_body(block_q, nch, d_ref, st_ref, dist_ref, minis_ref):
    d = d_ref[...]
    st = st_ref[...]
    d_sq = jnp.sum(d * d, axis=1, keepdims=True)
    s_sq = jnp.sum(st * st, axis=0, keepdims=True)
    dot = lax.dot_general(d, st, (((1,), (0,)), ((), ())),
                          preferred_element_type=jnp.float32)
    dist = d_sq + s_sq - 2.0 * dot
    dist_ref[...] = dist
    minis_ref[...] = jnp.min(dist.reshape(block_q, nch, CH), axis=2)


def _merge(a, b):
    ka, va = a
    kb, vb = b
    rkb = lax.rev(kb, (0,))
    rvb = lax.rev(vb, (0,))
    take = ka <= rkb
    mk = jnp.minimum(ka, rkb)
    mv = jnp.where(take, va, rvb)
    return plsc.sort_key_val(mk, mv)


def _tree(pairs):
    while len(pairs) > 1:
        pairs = [_merge(pairs[2 * i], pairs[2 * i + 1])
                 for i in range(len(pairs) // 2)]
    return pairs[0]


def _sc_body(rows, nch, d_mask, mkeys,
             dist_ref, minis_ref, smt_ref, out_ref,
             smt_v, minis_v, out_v, hcm0, hcm1, idx0, idx1, cand0, cand1,
             hist_v, sem0, sem1, osem):
    nvpr = nch // L
    rpg = mkeys // LW
    wid = lax.axis_index("s") * 2 + lax.axis_index("c")
    base = wid * rows

    pltpu.sync_copy(smt_ref, smt_v)

    iota = lax.iota(jnp.int32, L)
    lane_lt8 = iota < NVAL

    def row_fire(b, rl, idx_r, cand_r, hcm_r, sem_r):
        pairs = [plsc.sort_key_val(minis_v[pl.ds(rl * nch + j * L, L)],
                                   iota + j * L)
                 for j in range(nvpr)]
        _mk, hc = _tree(pairs)
        hcm_r[...] = hc
        idx_r[...] = (base + b * RB + rl) * rpg + (hc >> 3)
        pltpu.async_copy(dist_ref.at[idx_r], cand_r, sem_r)
        return hc

    def row_proc(r, cand_r, hcm_r, hc):
        hco = (hc & 7) * CH
        epairs = []
        for s in range(K):
            keys = cand_r[s, pl.ds(hco[s], CH)]
            epairs.append(plsc.sort_key_val(keys, iota + s * L))
        _ek, p = _tree(epairs)

        chunk = plsc.load_gather(hcm_r, [p >> 4])
        orig = chunk * CH + (p & (CH - 1))

        mode_vec = jnp.zeros((L,), jnp.float32)
        for d in range(d_mask):
            vals = plsc.load_gather(
                smt_v, [jnp.full((L,), d, jnp.int32), orig])
            vi = vals.astype(jnp.int32)
            hist_v[...] = jnp.zeros((L,), jnp.float32)
            plsc.addupdate_scatter(hist_v, [vi], jnp.ones((L,), jnp.float32))
            cnt = hist_v[...].astype(jnp.int32)
            score = cnt * NVAL + (NVAL - 1 - iota)
            best = jnp.max(score)
            mode = (NVAL - 1 - best % NVAL).astype(jnp.float32)
            mode_vec = jnp.where(iota == d, mode, mode_vec)

        plsc.store_scatter(out_v, [r * NVAL + iota], mode_vec,
                           mask=lane_lt8)

    def blk_body(b, carry0):
        pltpu.sync_copy(
            minis_ref.at[pl.ds((base + b * RB) * nch, RB * nch)], minis_v)

        hc_first = row_fire(b, 0, idx0, cand0, hcm0, sem0)

        def body2(i, hca):
            rl0 = 2 * i
            hcb = row_fire(b, rl0 + 1, idx1, cand1, hcm1, sem1)
            pltpu.make_async_copy(dist_ref.at[idx0], cand0, sem0).wait()
            row_proc(b * RB + rl0, cand0, hcm0, hca)
            nxt = jnp.minimum(rl0 + 2, RB - 1)
            hca2 = row_fire(b, nxt, idx0, cand0, hcm0, sem0)
            pltpu.make_async_copy(dist_ref.at[idx1], cand1, sem1).wait()
            row_proc(b * RB + rl0 + 1, cand1, hcm1, hcb)
            return hca2

        lax.fori_loop(0, RB // 2, body2, hc_first)
        pltpu.make_async_copy(dist_ref.at[idx0], cand0, sem0).wait()
        return carry0

    lax.fori_loop(0, rows // RB, blk_body, None)
    pltpu.async_copy(out_v, out_ref.at[pl.ds(base * NVAL, rows * NVAL)],
                     osem).wait()


def kernel(s_coor, s_mask, d_coor):
    mkeys, dim = s_coor.shape
    n = d_coor.shape[0]
    d_mask = s_mask.shape[1]
    nch = mkeys // CH
    block_q = 128 if n % 128 == 0 else n
    grid = n // block_q
    rows = n // NTILES

    s_t = s_coor.T
    smt = s_mask.T

    dist, minis = pl.pallas_call(
        functools.partial(_tc_body, block_q, nch),
        grid=(grid,),
        in_specs=[
            pl.BlockSpec((block_q, dim), lambda i: (i, 0)),
            pl.BlockSpec((dim, mkeys), lambda i: (0, 0)),
        ],
        out_specs=[
            pl.BlockSpec((block_q, mkeys), lambda i: (i, 0)),
            pl.BlockSpec((block_q, nch), lambda i: (i, 0)),
        ],
        out_shape=[
            jax.ShapeDtypeStruct((n, mkeys), jnp.float32),
            jax.ShapeDtypeStruct((n, nch), jnp.float32),
        ],
    )(d_coor, s_t)

    dist_rows = dist.reshape(n * mkeys // LW, LW)
    minis_flat = minis.reshape(n * nch)

    mesh = plsc.VectorSubcoreMesh(core_axis_name="c", subcore_axis_name="s",
                                  num_cores=2, num_subcores=16)
    sc = pl.kernel(
        functools.partial(_sc_body, rows, nch, d_mask, mkeys),
        out_type=jax.ShapeDtypeStruct((n * NVAL,), jnp.float32),
        mesh=mesh,
        compiler_params=pltpu.CompilerParams(needs_layout_passes=False,
                                             use_tc_tiling_on_sc=False),
        scratch_types=[
            pltpu.VMEM((d_mask, mkeys), jnp.float32),
            pltpu.VMEM((RB * nch,), jnp.float32),
            pltpu.VMEM((rows * NVAL,), jnp.float32),
            pltpu.VMEM((K,), jnp.int32),
            pltpu.VMEM((K,), jnp.int32),
            pltpu.VMEM((K,), jnp.int32),
            pltpu.VMEM((K,), jnp.int32),
            pltpu.VMEM((K, LW), jnp.float32),
            pltpu.VMEM((K, LW), jnp.float32),
            pltpu.VMEM((L,), jnp.float32),
            pltpu.SemaphoreType.DMA,
            pltpu.SemaphoreType.DMA,
            pltpu.SemaphoreType.DMA,
        ],
    )
    out = sc(dist_rows, minis_flat, smt)
    return out.reshape(n, NVAL)
